# baseline (speedup 1.0000x reference)
import functools

import jax
import jax.numpy as jnp
from jax import lax
from jax.experimental import pallas as pl
from jax.experimental.pallas import tpu as pltpu
from jax.experimental.pallas import tpu_sc as plsc

_N = 262144
_D = 64
_C = 16
_B = 8
_MARGIN = 0.5
_LOSS_W = 0.01

_NC = 2
_NS = 16
_NW = _NC * _NS
_SEG = _B * _C
_CHUNK = 512

_TCBLK = 8192
_TCGRID = _N // _TCBLK
_TCSPLIT = 24
_NSC = _TCSPLIT * _TCBLK
_PTS = _NSC // _NW
_NCHUNK = _PTS // _CHUNK


def _sc_body(target_hbm, batch_hbm, featT_hbm,
             out_cnt, out_sq,
             featT_buf, tgt_buf, bat_buf, cnt2_buf, sqa_buf, load_sem):
  cid = lax.axis_index("c")
  sid = lax.axis_index("s")
  wid = sid * _NC + cid
  base = wid * _PTS

  pltpu.sync_copy(target_hbm.at[pl.ds(base, _PTS)], tgt_buf)
  pltpu.sync_copy(batch_hbm.at[pl.ds(base, _PTS)], bat_buf)

  zf = jnp.zeros((16,), jnp.float32)
  zi = jnp.zeros((16,), jnp.int32)

  def zacc(r, carry):
    for k in range(_SEG // 16):
      cnt2_buf[r, pl.ds(k * 16, 16)] = zi
      sqa_buf[r, pl.ds(k * 16, 16)] = zf
    return 0
  lax.fori_loop(0, 16, zacc, 0)

  lane = lax.iota(jnp.int32, 16)
  ones16 = jnp.ones((16,), jnp.int32)

  pltpu.async_copy(featT_hbm.at[:, pl.ds(base, _CHUNK)],
                   featT_buf.at[0], load_sem)

  def chunk_body(g, carry):
    p = lax.bitwise_and(g, 1)
    off = g * _CHUNK
    pltpu.make_async_copy(featT_hbm.at[:, pl.ds(base + off, _CHUNK)],
                          featT_buf.at[p], load_sem).wait()

    @pl.when(g + 1 < _NCHUNK)
    def _():
      pltpu.async_copy(featT_hbm.at[:, pl.ds(base + off + _CHUNK, _CHUNK)],
                       featT_buf.at[1 - p], load_sem)

    def group_body(i, carry):
      t16 = tgt_buf[pl.ds(off + i * 16, 16)]
      b16 = bat_buf[pl.ds(off + i * 16, 16)]
      seg = b16 * _C + t16
      plsc.addupdate_scatter(cnt2_buf, [lane, seg], ones16)
      a0 = zf
      a1 = zf
      a2 = zf
      a3 = zf
      for d in range(0, _D, 4):
        v0 = featT_buf[p, d, pl.ds(i * 16, 16)]
        v1 = featT_buf[p, d + 1, pl.ds(i * 16, 16)]
        v2 = featT_buf[p, d + 2, pl.ds(i * 16, 16)]
        v3 = featT_buf[p, d + 3, pl.ds(i * 16, 16)]
        a0 = a0 + v0 * v0
        a1 = a1 + v1 * v1
        a2 = a2 + v2 * v2
        a3 = a3 + v3 * v3
      plsc.addupdate_scatter(sqa_buf, [lane, seg], (a0 + a1) + (a2 + a3))
      return 0
    lax.fori_loop(0, _CHUNK // 16, group_body, 0)
    return 0

  lax.fori_loop(0, _NCHUNK, chunk_body, 0)

  pltpu.sync_copy(cnt2_buf, out_cnt.at[wid])
  pltpu.sync_copy(sqa_buf, out_sq.at[wid])


def _sc_pass(target, featT, batch):
  mesh = plsc.VectorSubcoreMesh(core_axis_name="c", subcore_axis_name="s",
                                num_cores=_NC, num_subcores=_NS)
  f = pl.kernel(
      _sc_body,
      out_type=(
          jax.ShapeDtypeStruct((_NW, 16, _SEG), jnp.int32),
          jax.ShapeDtypeStruct((_NW, 16, _SEG), jnp.float32),
      ),
      mesh=mesh,
      compiler_params=pltpu.CompilerParams(needs_layout_passes=False,
                                           use_tc_tiling_on_sc=True),
      scratch_types=[
          pltpu.VMEM((2, _D, _CHUNK), jnp.float32),
          pltpu.VMEM((_PTS,), jnp.int32),
          pltpu.VMEM((_PTS,), jnp.int32),
          pltpu.VMEM((16, _SEG), jnp.int32),
          pltpu.VMEM((16, _SEG), jnp.float32),
          pltpu.SemaphoreType.DMA,
      ],
  )
  return f(target, batch, featT)


def _fs_body(tgt_ref, bat_ref, featT_ref, out_ref, out2_ref):
  i = pl.program_id(0)

  @pl.when(i == 0)
  def _():
    out_ref[...] = jnp.zeros((_D, _SEG), jnp.float32)
    out2_ref[...] = jnp.zeros((8, _SEG), jnp.float32)

  t = tgt_ref[0, 0, :]
  b = bat_ref[0, 0, :]
  trow = t.reshape(1, _TCBLK)
  brow = b.reshape(1, _TCBLK)
  cls = lax.broadcasted_iota(jnp.int32, (_C, _TCBLK), 0)
  f = featT_ref[...]
  tail = i >= _TCSPLIT

  def _mk_ext():
    sq2 = (f * f).sum(axis=0).reshape(1, _TCBLK)
    ones_r = jnp.ones((1, _TCBLK), jnp.float32)
    zeros_r = jnp.zeros((6, _TCBLK), jnp.float32)
    return jnp.concatenate([sq2, ones_r, zeros_r], axis=0)

  ext = lax.cond(tail, _mk_ext,
                 lambda: jnp.zeros((8, _TCBLK), jnp.float32))

  bmin = b[0]
  bmax = b[_TCBLK - 1]
  for bv in range(_B):
    @pl.when(jnp.logical_and(bv >= bmin, bv <= bmax))
    def _():
      ohb = jnp.where(jnp.logical_and(cls == trow, brow == bv), 1.0, 0.0)
      out_ref[:, bv * _C:(bv + 1) * _C] += lax.dot_general(
          f, ohb, (((1,), (1,)), ((), ())),
          preferred_element_type=jnp.float32)

      @pl.when(tail)
      def _():
        out2_ref[:, bv * _C:(bv + 1) * _C] += lax.dot_general(
            ext, ohb, (((1,), (1,)), ((), ())),
            preferred_element_type=jnp.float32)


def _fs_pass(target, batch, featT):
  t3 = target.reshape(_TCGRID, 1, _TCBLK)
  b3 = batch.reshape(_TCGRID, 1, _TCBLK)
  return pl.pallas_call(
      _fs_body,
      grid=(_TCGRID,),
      in_specs=[
          pl.BlockSpec((1, 1, _TCBLK), lambda i: (i, 0, 0)),
          pl.BlockSpec((1, 1, _TCBLK), lambda i: (i, 0, 0)),
          pl.BlockSpec((_D, _TCBLK), lambda i: (0, i)),
      ],
      out_specs=[
          pl.BlockSpec((_D, _SEG), lambda i: (0, 0)),
          pl.BlockSpec((8, _SEG), lambda i: (0, 0)),
      ],
      out_shape=(
          jax.ShapeDtypeStruct((_D, _SEG), jnp.float32),
          jax.ShapeDtypeStruct((8, _SEG), jnp.float32),
      ),
  )(t3, b3, featT)


def _epi_body(fsT_ref, sq_ref, cnt_ref, ext_ref, centT_ref, out_ref):
  f32 = jnp.float32
  fsT = fsT_ref[...]
  ext = ext_ref[...]
  sq_seg = sq_ref[...].sum(axis=(0, 1)) + ext[0]
  cnt_s = cnt_ref[...].sum(axis=(0, 1)).astype(f32) + ext[1]
  centT = centT_ref[...]
  centT_s = jnp.concatenate([centT] * _B, axis=1)

  ri = lax.broadcasted_iota(jnp.int32, (_SEG, _SEG), 0)
  ci = lax.broadcasted_iota(jnp.int32, (_SEG, _SEG), 1)
  idmat = (ri == ci).astype(f32)
  same = (lax.shift_right_logical(ri, 4) ==
          lax.shift_right_logical(ci, 4)).astype(f32)
  ones_mat = jnp.ones((_SEG, _SEG), f32)

  def bb(v):
    return (same * v[None, :]).sum(axis=1)

  def dotm(a, b):
    return lax.dot_general(a, b, (((1,), (0,)), ((), ())),
                           preferred_element_type=f32)

  cn2 = (centT_s * centT_s).sum(axis=0)
  dot_s = (fsT * centT_s).sum(axis=0)
  u = sq_seg - 2.0 * dot_s + cnt_s * cn2
  cnt_bb = bb(cnt_s)
  u_bb = bb(u)
  intra_c = jnp.where(cnt_bb > 0, u_bb / jnp.maximum(cnt_bb, 1.0), 0.0)
  total_intra = intra_c.sum() / _C
  batch_count = jnp.where(cnt_bb > 0, 1.0, 0.0).sum() / _C

  inv = 1.0 / jnp.maximum(cnt_s, 1.0)
  clscT = fsT * inv[None, :]
  gram = lax.dot_general(clscT, clscT, (((0,), (0,)), ((), ())),
                         preferred_element_type=f32)
  n_diag = gram * idmat
  n_rows = dotm(n_diag, ones_mat)
  n_cols = dotm(ones_mat, n_diag)
  sq = n_rows + n_cols - 2.0 * gram
  pos = sq > 0
  dist = jnp.where(pos, jnp.sqrt(jnp.where(pos, sq, 1.0)), 0.0)

  present = jnp.where(cnt_s > 0, 1.0, 0.0)
  d_pres = idmat * present[None, :]
  pres_rows = dotm(d_pres, ones_mat)
  pair_mask = pres_rows * present[None, :] * same * (1.0 - idmat)
  hinge = jnp.maximum(_MARGIN - dist, 0.0)
  hp_row = (hinge * pair_mask).sum(axis=1)
  npair_row = pair_mask.sum(axis=1)
  npair_bb = bb(npair_row)
  npres_bb = bb(present)
  inter_c = jnp.where(npres_bb > 1, hp_row / jnp.maximum(npair_bb, 1.0), 0.0)
  total_inter = inter_c.sum()

  avg_intra = jnp.where(batch_count > 0, total_intra / jnp.maximum(batch_count, 1.0), 0.0)
  avg_inter = jnp.where(batch_count > 0, total_inter / jnp.maximum(batch_count, 1.0), 0.0)
  out_ref[0, 0] = _LOSS_W * (avg_intra + avg_inter)


def _epilogue(fsT, sq, cnt, ext, centT):
  return pl.pallas_call(
      _epi_body,
      out_shape=jax.ShapeDtypeStruct((1, 1), jnp.float32),
      out_specs=pl.BlockSpec(memory_space=pltpu.SMEM),
  )(fsT, sq, cnt, ext, centT)


def kernel(pred, target, feat, batch, centers):
  featT = feat.T
  cnt, sq = _sc_pass(target, featT, batch)
  fsT, ext = _fs_pass(target, batch, featT)
  loss = _epilogue(fsT, sq, cnt, ext, centers.T)
  return loss[0, 0]


# speedup vs baseline: 1.1202x; 1.1202x over previous
import functools

import jax
import jax.numpy as jnp
from jax import lax
from jax.experimental import pallas as pl
from jax.experimental.pallas import tpu as pltpu
from jax.experimental.pallas import tpu_sc as plsc

_N = 262144
_D = 64
_C = 16
_B = 8
_MARGIN = 0.5
_LOSS_W = 0.01

_NC = 2
_NS = 16
_NW = _NC * _NS
_SEG = _B * _C
_CHUNK = 512

_TCBLK = 32768
_TCGRID = _N // _TCBLK
_TCSPLIT = 6
_NSC = _TCSPLIT * _TCBLK
_PTS = _NSC // _NW
_NCHUNK = _PTS // _CHUNK


def _sc_body(target_hbm, batch_hbm, featT_hbm,
             out_cnt, out_sq,
             featT_buf, tgt_buf, bat_buf, cnt2_buf, sqa_buf, load_sem):
  cid = lax.axis_index("c")
  sid = lax.axis_index("s")
  wid = sid * _NC + cid
  base = wid * _PTS

  pltpu.sync_copy(target_hbm.at[pl.ds(base, _PTS)], tgt_buf)
  pltpu.sync_copy(batch_hbm.at[pl.ds(base, _PTS)], bat_buf)

  zf = jnp.zeros((16,), jnp.float32)
  zi = jnp.zeros((16,), jnp.int32)

  def zacc(r, carry):
    for k in range(_SEG // 16):
      cnt2_buf[r, pl.ds(k * 16, 16)] = zi
      sqa_buf[r, pl.ds(k * 16, 16)] = zf
    return 0
  lax.fori_loop(0, 16, zacc, 0)

  lane = lax.iota(jnp.int32, 16)
  ones16 = jnp.ones((16,), jnp.int32)

  pltpu.async_copy(featT_hbm.at[:, pl.ds(base, _CHUNK)],
                   featT_buf.at[0], load_sem)

  def chunk_body(g, carry):
    p = lax.bitwise_and(g, 1)
    off = g * _CHUNK
    pltpu.make_async_copy(featT_hbm.at[:, pl.ds(base + off, _CHUNK)],
                          featT_buf.at[p], load_sem).wait()

    @pl.when(g + 1 < _NCHUNK)
    def _():
      pltpu.async_copy(featT_hbm.at[:, pl.ds(base + off + _CHUNK, _CHUNK)],
                       featT_buf.at[1 - p], load_sem)

    def group_body(i, carry):
      t16 = tgt_buf[pl.ds(off + i * 16, 16)]
      b16 = bat_buf[pl.ds(off + i * 16, 16)]
      seg = b16 * _C + t16
      plsc.addupdate_scatter(cnt2_buf, [lane, seg], ones16)
      a0 = zf
      a1 = zf
      a2 = zf
      a3 = zf
      for d in range(0, _D, 4):
        v0 = featT_buf[p, d, pl.ds(i * 16, 16)]
        v1 = featT_buf[p, d + 1, pl.ds(i * 16, 16)]
        v2 = featT_buf[p, d + 2, pl.ds(i * 16, 16)]
        v3 = featT_buf[p, d + 3, pl.ds(i * 16, 16)]
        a0 = a0 + v0 * v0
        a1 = a1 + v1 * v1
        a2 = a2 + v2 * v2
        a3 = a3 + v3 * v3
      plsc.addupdate_scatter(sqa_buf, [lane, seg], (a0 + a1) + (a2 + a3))
      return 0
    lax.fori_loop(0, _CHUNK // 16, group_body, 0)
    return 0

  lax.fori_loop(0, _NCHUNK, chunk_body, 0)

  pltpu.sync_copy(cnt2_buf, out_cnt.at[wid])
  pltpu.sync_copy(sqa_buf, out_sq.at[wid])


def _sc_pass(target, featT, batch):
  mesh = plsc.VectorSubcoreMesh(core_axis_name="c", subcore_axis_name="s",
                                num_cores=_NC, num_subcores=_NS)
  f = pl.kernel(
      _sc_body,
      out_type=(
          jax.ShapeDtypeStruct((_NW, 16, _SEG), jnp.int32),
          jax.ShapeDtypeStruct((_NW, 16, _SEG), jnp.float32),
      ),
      mesh=mesh,
      compiler_params=pltpu.CompilerParams(needs_layout_passes=False,
                                           use_tc_tiling_on_sc=True),
      scratch_types=[
          pltpu.VMEM((2, _D, _CHUNK), jnp.float32),
          pltpu.VMEM((_PTS,), jnp.int32),
          pltpu.VMEM((_PTS,), jnp.int32),
          pltpu.VMEM((16, _SEG), jnp.int32),
          pltpu.VMEM((16, _SEG), jnp.float32),
          pltpu.SemaphoreType.DMA,
      ],
  )
  return f(target, batch, featT)


def _fs_body(tgt_ref, bat_ref, featT_ref, out_ref, out2_ref):
  i = pl.program_id(0)

  @pl.when(i == 0)
  def _():
    out_ref[...] = jnp.zeros((_D, _SEG), jnp.float32)
    out2_ref[...] = jnp.zeros((8, _SEG), jnp.float32)

  t = tgt_ref[0, 0, :]
  b = bat_ref[0, 0, :]
  trow = t.reshape(1, _TCBLK)
  brow = b.reshape(1, _TCBLK)
  cls = lax.broadcasted_iota(jnp.int32, (_C, _TCBLK), 0)
  f = featT_ref[...]
  tail = i >= _TCSPLIT

  def _mk_ext():
    sq2 = (f * f).sum(axis=0).reshape(1, _TCBLK)
    ones_r = jnp.ones((1, _TCBLK), jnp.float32)
    zeros_r = jnp.zeros((6, _TCBLK), jnp.float32)
    return jnp.concatenate([sq2, ones_r, zeros_r], axis=0)

  ext = lax.cond(tail, _mk_ext,
                 lambda: jnp.zeros((8, _TCBLK), jnp.float32))

  bmin = b[0]
  bmax = b[_TCBLK - 1]
  for bv in range(_B):
    @pl.when(jnp.logical_and(bv >= bmin, bv <= bmax))
    def _():
      ohb = jnp.where(jnp.logical_and(cls == trow, brow == bv), 1.0, 0.0)
      out_ref[:, bv * _C:(bv + 1) * _C] += lax.dot_general(
          f, ohb, (((1,), (1,)), ((), ())),
          preferred_element_type=jnp.float32)

      @pl.when(tail)
      def _():
        out2_ref[:, bv * _C:(bv + 1) * _C] += lax.dot_general(
            ext, ohb, (((1,), (1,)), ((), ())),
            preferred_element_type=jnp.float32)


def _fs_pass(target, batch, featT):
  t3 = target.reshape(_TCGRID, 1, _TCBLK)
  b3 = batch.reshape(_TCGRID, 1, _TCBLK)
  return pl.pallas_call(
      _fs_body,
      grid=(_TCGRID,),
      in_specs=[
          pl.BlockSpec((1, 1, _TCBLK), lambda i: (i, 0, 0)),
          pl.BlockSpec((1, 1, _TCBLK), lambda i: (i, 0, 0)),
          pl.BlockSpec((_D, _TCBLK), lambda i: (0, i)),
      ],
      out_specs=[
          pl.BlockSpec((_D, _SEG), lambda i: (0, 0)),
          pl.BlockSpec((8, _SEG), lambda i: (0, 0)),
      ],
      out_shape=(
          jax.ShapeDtypeStruct((_D, _SEG), jnp.float32),
          jax.ShapeDtypeStruct((8, _SEG), jnp.float32),
      ),
  )(t3, b3, featT)


def _epi_body(fsT_ref, sq_ref, cnt_ref, ext_ref, centT_ref, out_ref):
  f32 = jnp.float32
  fsT = fsT_ref[...]
  ext = ext_ref[...]
  sq_seg = sq_ref[...].sum(axis=(0, 1)) + ext[0]
  cnt_s = cnt_ref[...].sum(axis=(0, 1)).astype(f32) + ext[1]
  centT = centT_ref[...]
  centT_s = jnp.concatenate([centT] * _B, axis=1)

  ri = lax.broadcasted_iota(jnp.int32, (_SEG, _SEG), 0)
  ci = lax.broadcasted_iota(jnp.int32, (_SEG, _SEG), 1)
  idmat = (ri == ci).astype(f32)
  same = (lax.shift_right_logical(ri, 4) ==
          lax.shift_right_logical(ci, 4)).astype(f32)
  ones_mat = jnp.ones((_SEG, _SEG), f32)

  def bb(v):
    return (same * v[None, :]).sum(axis=1)

  def dotm(a, b):
    return lax.dot_general(a, b, (((1,), (0,)), ((), ())),
                           preferred_element_type=f32)

  cn2 = (centT_s * centT_s).sum(axis=0)
  dot_s = (fsT * centT_s).sum(axis=0)
  u = sq_seg - 2.0 * dot_s + cnt_s * cn2
  cnt_bb = bb(cnt_s)
  u_bb = bb(u)
  intra_c = jnp.where(cnt_bb > 0, u_bb / jnp.maximum(cnt_bb, 1.0), 0.0)
  total_intra = intra_c.sum() / _C
  batch_count = jnp.where(cnt_bb > 0, 1.0, 0.0).sum() / _C

  inv = 1.0 / jnp.maximum(cnt_s, 1.0)
  clscT = fsT * inv[None, :]
  gram = lax.dot_general(clscT, clscT, (((0,), (0,)), ((), ())),
                         preferred_element_type=f32)
  n_diag = gram * idmat
  n_rows = dotm(n_diag, ones_mat)
  n_cols = dotm(ones_mat, n_diag)
  sq = n_rows + n_cols - 2.0 * gram
  pos = sq > 0
  dist = jnp.where(pos, jnp.sqrt(jnp.where(pos, sq, 1.0)), 0.0)

  present = jnp.where(cnt_s > 0, 1.0, 0.0)
  d_pres = idmat * present[None, :]
  pres_rows = dotm(d_pres, ones_mat)
  pair_mask = pres_rows * present[None, :] * same * (1.0 - idmat)
  hinge = jnp.maximum(_MARGIN - dist, 0.0)
  hp_row = (hinge * pair_mask).sum(axis=1)
  npair_row = pair_mask.sum(axis=1)
  npair_bb = bb(npair_row)
  npres_bb = bb(present)
  inter_c = jnp.where(npres_bb > 1, hp_row / jnp.maximum(npair_bb, 1.0), 0.0)
  total_inter = inter_c.sum()

  avg_intra = jnp.where(batch_count > 0, total_intra / jnp.maximum(batch_count, 1.0), 0.0)
  avg_inter = jnp.where(batch_count > 0, total_inter / jnp.maximum(batch_count, 1.0), 0.0)
  out_ref[0, 0] = _LOSS_W * (avg_intra + avg_inter)


def _epilogue(fsT, sq, cnt, ext, centT):
  return pl.pallas_call(
      _epi_body,
      out_shape=jax.ShapeDtypeStruct((1, 1), jnp.float32),
      out_specs=pl.BlockSpec(memory_space=pltpu.SMEM),
  )(fsT, sq, cnt, ext, centT)


def kernel(pred, target, feat, batch, centers):
  featT = feat.T
  cnt, sq = _sc_pass(target, featT, batch)
  fsT, ext = _fs_pass(target, batch, featT)
  loss = _epilogue(fsT, sq, cnt, ext, centers.T)
  return loss[0, 0]
